# baseline (speedup 1.0000x reference)
import jax
import jax.numpy as jnp
from jax import lax
from jax.experimental import pallas as pl
from jax.experimental.pallas import tpu as pltpu
from jax.experimental.pallas import tpu_sc as plsc

_B = 4096
_L = 200
_LP = 208
_DIM = 64
_NTILES = 32
_UPT = _B // _NTILES
_NG = _LP // 16


def _tec_body(uf_hbm, if_hbm, ub_hbm, ib_hbm, user_hbm, item_hbm, mask_hbm,
              out_hbm,
              uidx_v, ufac_v, ubias_v,
              iidx_v, rows_v, ibias_v, mask_v, out_v,
              sem_rows, sem_misc):
    nc = 2
    wid = lax.axis_index("s") * nc + lax.axis_index("c")
    base = wid * _UPT

    pltpu.sync_copy(user_hbm.at[pl.ds(base, _UPT)], uidx_v)
    pltpu.async_copy(uf_hbm.at[uidx_v], ufac_v, sem_misc).wait()
    pltpu.async_copy(ub_hbm.at[uidx_v], ubias_v.at[pl.ds(0, _UPT)],
                     sem_misc).wait()

    @pl.loop(0, _UPT)
    def _user_loop(i):
        b = base + i
        pltpu.sync_copy(item_hbm.at[b], iidx_v.at[pl.ds(0, _L)])
        pltpu.sync_copy(mask_hbm.at[b], mask_v.at[pl.ds(0, _L)])
        c0 = pltpu.async_copy(if_hbm.at[iidx_v.at[pl.ds(0, 128)]],
                              rows_v.at[pl.ds(0, 128)], sem_rows)
        c1 = pltpu.async_copy(if_hbm.at[iidx_v.at[pl.ds(128, 72)]],
                              rows_v.at[pl.ds(128, 72)], sem_rows)
        c2 = pltpu.async_copy(ib_hbm.at[iidx_v.at[pl.ds(0, 128)]],
                              ibias_v.at[pl.ds(0, 128)], sem_misc)
        c3 = pltpu.async_copy(ib_hbm.at[iidx_v.at[pl.ds(128, 72)]],
                              ibias_v.at[pl.ds(128, 72)], sem_misc)
        c0.wait()
        c1.wait()
        c2.wait()
        c3.wait()

        ub = ubias_v[pl.ds(i, 16)][0]
        u_vecs = [ufac_v[i, pl.ds(16 * k, 16)] for k in range(4)]

        lanes = lax.iota(jnp.int32, 16)
        row_ids = [lanes + (16 * g) for g in range(_NG)]
        accs = [ibias_v[pl.ds(pl.multiple_of(16 * g, 16), 16)] + ub
                for g in range(_NG)]
        cols = jnp.zeros((16,), jnp.int32)
        for h in range(_DIM):
            us = u_vecs[h // 16][h % 16]
            for g in range(_NG):
                vals = plsc.load_gather(rows_v, [row_ids[g], cols])
                accs[g] = accs[g] + vals * us
            cols = cols + 1

        for g in range(_NG):
            off = pl.multiple_of(16 * g, 16)
            m = mask_v[pl.ds(off, 16)]
            out_v[pl.ds(off, 16)] = jnp.where(
                m == 0, jnp.float32(-1e13), accs[g])

        pltpu.sync_copy(out_v.at[pl.ds(0, _L)], out_hbm.at[b])


@jax.jit
def kernel(user_factors, item_factors, user_biases, item_biases,
           user, item, mask):
    mesh = plsc.VectorSubcoreMesh(core_axis_name="c", subcore_axis_name="s")
    run = pl.kernel(
        _tec_body,
        out_type=jax.ShapeDtypeStruct((_B, _L), jnp.float32),
        mesh=mesh,
        scratch_types=[
            pltpu.VMEM((_UPT,), jnp.int32),
            pltpu.VMEM((_UPT, _DIM), jnp.float32),
            pltpu.VMEM((_UPT + 16,), jnp.float32),
            pltpu.VMEM((_LP,), jnp.int32),
            pltpu.VMEM((_LP, _DIM), jnp.float32),
            pltpu.VMEM((_LP,), jnp.float32),
            pltpu.VMEM((_LP,), jnp.int32),
            pltpu.VMEM((_LP,), jnp.float32),
            pltpu.SemaphoreType.DMA,
            pltpu.SemaphoreType.DMA,
        ],
        compiler_params=pltpu.CompilerParams(
            needs_layout_passes=False, use_tc_tiling_on_sc=False),
    )
    return run(user_factors, item_factors, user_biases, item_biases,
               user.astype(jnp.int32), item.astype(jnp.int32), mask)
